# baseline (speedup 1.0000x reference)
import functools

import jax
import jax.numpy as jnp
from jax import lax
from jax.experimental import pallas as pl
from jax.experimental.pallas import tpu as pltpu
from jax.experimental.pallas import tpu_sc as plsc

_N = 10000
_NP = 10240
_K = 20
_C = 512
_NCHUNK = _NP // _C
_RB = 400
_BN = 400
_E = _N * _K
_NW = 32
_CB = 128

_N1 = 8000
_N2 = 2000
_NP1 = 8192
_NP2 = 2048

_BIG2 = 3.0e38
_BIGF = 1.0e9


def _make_knn_body(base):
    def _knn_body(lo_ref, hi_ref, g_ref, gt_ref, brow_ref, bcol_ref, nbr_ref,
                  dist_ref):
        i = pl.program_id(0)
        g = g_ref[...]
        row_ids = base + i * _RB + lax.broadcasted_iota(jnp.int32, (_RB, 1),
                                                        0)
        brow = brow_ref[...]
        lo = lo_ref[i]
        hi = hi_ref[i]
        clo = lo // _C
        chi = (hi + _C - 1) // _C

        def dist_body(c, _):
            gt = gt_ref[c]
            d = -2.0 * jnp.dot(g, gt, preferred_element_type=jnp.float32)
            sq_c = jnp.sum(gt * gt, axis=0, keepdims=True)
            d = d + sq_c
            bcol = bcol_ref[c]
            col_ids = c * _C + lax.broadcasted_iota(jnp.int32, (_RB, _C), 1)
            bad = (brow != bcol) | (col_ids == row_ids)
            dist_ref[c] = jnp.where(bad, jnp.inf, d)
            return 0

        lax.fori_loop(clo, chi, dist_body, 0)

        colf_base = lax.broadcasted_iota(jnp.int32, (_RB, _C),
                                         1).astype(jnp.float32)
        i_prev = jnp.full((_RB, 1), -1.0, jnp.float32)
        for k in range(_K):
            def sel_body(c, carry, i_prev=i_prev):
                bv, bi = carry
                colf = (c * _C).astype(jnp.float32) + colf_base
                d = jnp.where(colf == i_prev, _BIG2, dist_ref[c])
                dist_ref[c] = d
                cmin = jnp.min(d, axis=1, keepdims=True)
                cidx = jnp.min(jnp.where(d == cmin, colf, _BIGF),
                               axis=1, keepdims=True)
                take = (cmin < bv) | ((cmin == bv) & (cidx < bi))
                return jnp.where(take, cmin, bv), jnp.where(take, cidx, bi)

            best_v, best_i = lax.fori_loop(
                clo, chi, sel_body,
                (jnp.full((_RB, 1), _BIG2, jnp.float32),
                 jnp.full((_RB, 1), -1.0, jnp.float32)))
            nbr_ref[:, k:k + 1] = jnp.clip(best_i, 0.0,
                                           float(_N - 1)).astype(jnp.int32)
            i_prev = best_i

    return _knn_body


def _knn(lo, hi, geom_rows, gt, brow, bcol, base, nrows):
    return pl.pallas_call(
        _make_knn_body(base),
        grid=(nrows // _RB,),
        in_specs=[
            pl.BlockSpec(memory_space=pltpu.SMEM),
            pl.BlockSpec(memory_space=pltpu.SMEM),
            pl.BlockSpec((_RB, 128), lambda i: (i, 0)),
            pl.BlockSpec((_NCHUNK, 128, _C), lambda i: (0, 0, 0)),
            pl.BlockSpec((_RB, 1), lambda i: (i, 0)),
            pl.BlockSpec((_NCHUNK, 1, _C), lambda i: (0, 0, 0)),
        ],
        out_specs=pl.BlockSpec((_RB, 128), lambda i: (i, 0)),
        out_shape=jax.ShapeDtypeStruct((nrows, 128), jnp.int32),
        scratch_shapes=[pltpu.VMEM((_NCHUNK, _RB, _C), jnp.float32)],
    )(lo, hi, geom_rows, gt, brow, bcol)


def _prep_body(g_ref, aux_ref, w1_ref, b1_ref, p_ref, tab_ref):
    wa = w1_ref[0:128, :]
    wb = w1_ref[128:256, :]
    g = g_ref[...]
    p_ref[...] = (jnp.dot(g, wa - wb, preferred_element_type=jnp.float32)
                  + b1_ref[...])
    q = jnp.dot(g, wb, preferred_element_type=jnp.float32)
    tab_ref[...] = jnp.concatenate(
        [q, aux_ref[...], jnp.zeros((_N, 112), jnp.float32)], axis=1)


def _prep(geom, aux, W1, b1r):
    return pl.pallas_call(
        _prep_body,
        out_shape=(jax.ShapeDtypeStruct((_N, 128), jnp.float32),
                   jax.ShapeDtypeStruct((_N, 256), jnp.float32)),
    )(geom, aux, W1, b1r)


def _sc_gather_call(tab, src, nch):
    mesh = plsc.VectorSubcoreMesh(core_axis_name="c", subcore_axis_name="s")

    @functools.partial(
        pl.kernel,
        mesh=mesh,
        out_type=jax.ShapeDtypeStruct((_NW, nch, _CB, 256), jnp.float32),
        scratch_types=[
            pltpu.VMEM((nch, _CB), jnp.int32),
            pltpu.VMEM((_CB, 256), jnp.float32),
            pltpu.VMEM((_CB, 256), jnp.float32),
            pltpu.SemaphoreType.DMA,
            pltpu.SemaphoreType.DMA,
        ],
    )
    def gather_kernel(tab_hbm, src_hbm, out_hbm, idx_v, buf0, buf1, sem0,
                      sem1):
        wid = lax.axis_index("s") * 2 + lax.axis_index("c")
        pltpu.sync_copy(src_hbm.at[wid], idx_v)
        bufs = (buf0, buf1)
        sems = (sem0, sem1)
        pltpu.async_copy(tab_hbm.at[idx_v.at[0]], buf0, sem0)

        @pl.loop(0, nch, step=2)
        def _(c):
            for b in range(2):
                ch = c + b
                nxt = ch + 1
                pltpu.make_async_copy(tab_hbm.at[idx_v.at[ch % nch]],
                                      bufs[b], sems[b]).wait()

                @pl.when(nxt < nch)
                def _():
                    pltpu.async_copy(tab_hbm.at[idx_v.at[nxt % nch]],
                                     bufs[1 - b], sems[1 - b])

                pltpu.sync_copy(bufs[b], out_hbm.at[wid, ch])

    return gather_kernel(tab, src)


def _stage1_body(qg_ref, p_ref, st_ref):
    i = pl.program_id(0)
    p = p_ref[...]
    s = jnp.zeros((1, 128), jnp.float32)
    ss = jnp.zeros((1, 128), jnp.float32)
    for k in range(_K):
        h = qg_ref[k] + p
        s = s + jnp.sum(h, axis=0, keepdims=True)
        ss = ss + jnp.sum(h * h, axis=0, keepdims=True)

    @pl.when(i == 0)
    def _():
        st_ref[...] = jnp.zeros_like(st_ref)

    st_ref[0:1, :] = st_ref[0:1, :] + s
    st_ref[1:2, :] = st_ref[1:2, :] + ss


def _stage1(eg, p, nblk):
    return pl.pallas_call(
        _stage1_body,
        grid=(nblk,),
        in_specs=[
            pl.BlockSpec((_K, _BN, 128), lambda i: (0, i, 0)),
            pl.BlockSpec((_BN, 128), lambda i: (i, 0)),
        ],
        out_specs=pl.BlockSpec((8, 128), lambda i: (0, 0)),
        out_shape=jax.ShapeDtypeStruct((8, 128), jnp.float32),
    )(eg, p)


def _stage2_body(qg_ref, p_ref, st1_ref, w2_ref, b2_ref, g1_ref, bb1_ref,
                 h2_ref, st2_ref):
    i = pl.program_id(0)
    inv_e = 1.0 / float(_E)
    m = st1_ref[0:1, :] * inv_e
    v = st1_ref[1:2, :] * inv_e - m * m
    sc = g1_ref[...] / jnp.sqrt(v + 1e-5)
    sh = bb1_ref[...] - m * sc
    p = p_ref[...]
    hcat = jnp.concatenate([qg_ref[k] + p for k in range(_K)], axis=0)
    h = jnp.maximum(hcat * sc + sh, 0.0)
    h2 = jnp.dot(h, w2_ref[...], preferred_element_type=jnp.float32) \
        + b2_ref[...]
    s = jnp.sum(h2, axis=0, keepdims=True)
    ss = jnp.sum(h2 * h2, axis=0, keepdims=True)
    for k in range(_K):
        h2_ref[k] = h2[k * _BN:(k + 1) * _BN, :]

    @pl.when(i == 0)
    def _():
        st2_ref[...] = jnp.zeros_like(st2_ref)

    st2_ref[0:1, :] = st2_ref[0:1, :] + s
    st2_ref[1:2, :] = st2_ref[1:2, :] + ss


def _stage2(eg, p, st1, W2, b2r, g1r, bb1r, nblk, npad):
    return pl.pallas_call(
        _stage2_body,
        grid=(nblk,),
        in_specs=[
            pl.BlockSpec((_K, _BN, 128), lambda i: (0, i, 0)),
            pl.BlockSpec((_BN, 128), lambda i: (i, 0)),
            pl.BlockSpec((8, 128), lambda i: (0, 0)),
            pl.BlockSpec((128, 128), lambda i: (0, 0)),
            pl.BlockSpec((1, 128), lambda i: (0, 0)),
            pl.BlockSpec((1, 128), lambda i: (0, 0)),
            pl.BlockSpec((1, 128), lambda i: (0, 0)),
        ],
        out_specs=(pl.BlockSpec((_K, _BN, 128), lambda i: (0, i, 0)),
                   pl.BlockSpec((8, 128), lambda i: (0, 0))),
        out_shape=(jax.ShapeDtypeStruct((_K, npad, 128), jnp.float32),
                   jax.ShapeDtypeStruct((8, 128), jnp.float32)),
    )(eg, p, st1, W2, b2r, g1r, bb1r)


def _stage3_body(h2_ref, ag_ref, aux_ref, st2_ref, wa1_ref, ba1_ref, wa2_ref,
                 ba2_ref, g2_ref, bb2_ref, lng_ref, lnb_ref, out_ref):
    inv_e = 1.0 / float(_E)
    m = st2_ref[0:1, :] * inv_e
    v = st2_ref[1:2, :] * inv_e - m * m
    sc2 = g2_ref[...] / jnp.sqrt(v + 1e-5)
    sh2 = bb2_ref[...] - m * sc2
    ai = aux_ref[...]
    ea = jnp.concatenate(
        [jnp.concatenate([ai, ag_ref[k][:, 0:16]], axis=1)
         for k in range(_K)], axis=0)
    a = jnp.maximum(
        jnp.dot(ea, wa1_ref[...], preferred_element_type=jnp.float32)
        + ba1_ref[...], 0.0)
    gb = jnp.dot(a, wa2_ref[...], preferred_element_type=jnp.float32) \
        + ba2_ref[...]
    o = jnp.full((_BN, 128), -jnp.inf, jnp.float32)
    for k in range(_K):
        ef = jnp.maximum(h2_ref[k] * sc2 + sh2, 0.0)
        gbk = gb[k * _BN:(k + 1) * _BN, :]
        gamma = 1.0 / (1.0 + jnp.exp(-(gbk[:, 0:128] + 1.0)))
        beta = gbk[:, 128:256]
        o = jnp.maximum(o, gamma * ef + beta)
    mu = jnp.mean(o, axis=1, keepdims=True)
    d = o - mu
    var = jnp.mean(d * d, axis=1, keepdims=True)
    out_ref[...] = jnp.maximum(
        d / jnp.sqrt(var + 1e-5) * lng_ref[...] + lnb_ref[...], 0.0)


def _stage3(h2, eg, aux_h, st2, Wa1, ba1r, Wa2, ba2r, g2r, bb2r, lngr, lnbr,
            nblk, nreal):
    return pl.pallas_call(
        _stage3_body,
        grid=(nblk,),
        in_specs=[
            pl.BlockSpec((_K, _BN, 128), lambda i: (0, i, 0)),
            pl.BlockSpec((_K, _BN, 128), lambda i: (0, i, 1)),
            pl.BlockSpec((_BN, 16), lambda i: (i, 0)),
            pl.BlockSpec((8, 128), lambda i: (0, 0)),
            pl.BlockSpec((32, 64), lambda i: (0, 0)),
            pl.BlockSpec((1, 64), lambda i: (0, 0)),
            pl.BlockSpec((64, 256), lambda i: (0, 0)),
            pl.BlockSpec((1, 256), lambda i: (0, 0)),
            pl.BlockSpec((1, 128), lambda i: (0, 0)),
            pl.BlockSpec((1, 128), lambda i: (0, 0)),
            pl.BlockSpec((1, 128), lambda i: (0, 0)),
            pl.BlockSpec((1, 128), lambda i: (0, 0)),
        ],
        out_specs=pl.BlockSpec((_BN, 128), lambda i: (i, 0)),
        out_shape=jax.ShapeDtypeStruct((nreal, 128), jnp.float32),
    )(h2, eg, aux_h, st2, Wa1, ba1r, Wa2, ba2r, g2r, bb2r, lngr, lnbr)


def _make_src(nbr, nreal, npad):
    pad_idx = jnp.broadcast_to(
        jnp.arange(npad - nreal, dtype=jnp.int32)[None, :],
        (_K, npad - nreal))
    src = jnp.concatenate([nbr[:, :_K].T, pad_idx], axis=1)
    return src.reshape(_NW, (npad * _K) // (_NW * _CB), _CB)


def kernel(geom, aux, batch, W1, b1, bn1_g, bn1_b, W2, b2, bn2_g, bn2_b,
           Wa1, ba1, Wa2, ba2, ln_g, ln_b):
    batch = batch.astype(jnp.int32)
    geom_pad = jnp.pad(geom, ((0, _NP - _N), (0, 0)))
    gt = geom_pad.T.reshape(128, _NCHUNK, _C).transpose(1, 0, 2)
    bcol = jnp.pad(batch, (0, _NP - _N),
                   constant_values=-2).reshape(_NCHUNK, 1, _C)
    brow_all = batch.reshape(_N, 1)

    def win(base, nrows):
        nblk = nrows // _RB
        first = batch[base + jnp.arange(nblk) * _RB]
        last = batch[base + (jnp.arange(nblk) + 1) * _RB - 1]
        lo = jnp.searchsorted(batch, first, side='left').astype(jnp.int32)
        hi = jnp.searchsorted(batch, last, side='right').astype(jnp.int32)
        return lo, hi

    lo1, hi1 = win(0, _N1)
    lo2, hi2 = win(_N1, _N2)

    nbr1 = _knn(lo1, hi1, geom[0:_N1], gt, brow_all[0:_N1], bcol, 0, _N1)
    nbr2 = _knn(lo2, hi2, geom[_N1:_N], gt, brow_all[_N1:_N], bcol, _N1, _N2)

    b1r = b1.reshape(1, 128)
    p, tab = _prep(geom, aux, W1, b1r)

    src1 = _make_src(nbr1, _N1, _NP1)
    src2 = _make_src(nbr2, _N2, _NP2)
    eg1 = _sc_gather_call(tab, src1, (_NP1 * _K) // (_NW * _CB))
    eg2 = _sc_gather_call(tab, src2, (_NP2 * _K) // (_NW * _CB))
    eg1 = eg1.reshape(_K, _NP1, 256)
    eg2 = eg2.reshape(_K, _NP2, 256)

    p1, p2 = p[0:_N1], p[_N1:_N]
    nblk1, nblk2 = _N1 // _BN, _N2 // _BN
    st1 = _stage1(eg1, p1, nblk1) + _stage1(eg2, p2, nblk2)
    b2r = b2.reshape(1, 128)
    g1r, bb1r = bn1_g.reshape(1, 128), bn1_b.reshape(1, 128)
    h2a, st2a = _stage2(eg1, p1, st1, W2, b2r, g1r, bb1r, nblk1, _NP1)
    h2b, st2b = _stage2(eg2, p2, st1, W2, b2r, g1r, bb1r, nblk2, _NP2)
    st2 = st2a + st2b

    s3 = (Wa1, ba1.reshape(1, 64), Wa2, ba2.reshape(1, 256),
          bn2_g.reshape(1, 128), bn2_b.reshape(1, 128),
          ln_g.reshape(1, 128), ln_b.reshape(1, 128))
    out1 = _stage3(h2a, eg1, aux[0:_N1], st2, *s3, nblk=nblk1, nreal=_N1)
    out2 = _stage3(h2b, eg2, aux[_N1:_N], st2, *s3, nblk=nblk2, nreal=_N2)
    return jnp.concatenate([out1, out2], axis=0)


# speedup vs baseline: 1.0329x; 1.0329x over previous
import functools

import jax
import jax.numpy as jnp
from jax import lax
from jax.experimental import pallas as pl
from jax.experimental.pallas import tpu as pltpu
from jax.experimental.pallas import tpu_sc as plsc

_N = 10000
_NP = 10240
_K = 20
_C = 512
_NCHUNK = _NP // _C
_RB = 400
_BN = 400
_E = _N * _K
_NW = 32
_CB = 128

_N1 = 6000
_N2 = 4000
_NP1 = 6144
_NP2 = 4096

_BIG2 = 3.0e38
_BIGF = 1.0e9


def _make_knn_body(base):
    def _knn_body(lo_ref, hi_ref, g_ref, gt_ref, brow_ref, bcol_ref, nbr_ref,
                  dist_ref):
        i = pl.program_id(0)
        g = g_ref[...]
        row_ids = base + i * _RB + lax.broadcasted_iota(jnp.int32, (_RB, 1),
                                                        0)
        brow = brow_ref[...]
        lo = lo_ref[i]
        hi = hi_ref[i]
        clo = lo // _C
        chi = (hi + _C - 1) // _C

        def dist_body(c, _):
            gt = gt_ref[c]
            d = -2.0 * jnp.dot(g, gt, preferred_element_type=jnp.float32)
            sq_c = jnp.sum(gt * gt, axis=0, keepdims=True)
            d = d + sq_c
            bcol = bcol_ref[c]
            col_ids = c * _C + lax.broadcasted_iota(jnp.int32, (_RB, _C), 1)
            bad = (brow != bcol) | (col_ids == row_ids)
            dist_ref[c] = jnp.where(bad, jnp.inf, d)
            return 0

        lax.fori_loop(clo, chi, dist_body, 0)

        colf_base = lax.broadcasted_iota(jnp.int32, (_RB, _C),
                                         1).astype(jnp.float32)
        i_prev = jnp.full((_RB, 1), -1.0, jnp.float32)
        for k in range(_K):
            def sel_body(c, carry, i_prev=i_prev):
                bv, bi = carry
                colf = (c * _C).astype(jnp.float32) + colf_base
                d = jnp.where(colf == i_prev, _BIG2, dist_ref[c])
                dist_ref[c] = d
                cmin = jnp.min(d, axis=1, keepdims=True)
                cidx = jnp.min(jnp.where(d == cmin, colf, _BIGF),
                               axis=1, keepdims=True)
                take = (cmin < bv) | ((cmin == bv) & (cidx < bi))
                return jnp.where(take, cmin, bv), jnp.where(take, cidx, bi)

            best_v, best_i = lax.fori_loop(
                clo, chi, sel_body,
                (jnp.full((_RB, 1), _BIG2, jnp.float32),
                 jnp.full((_RB, 1), -1.0, jnp.float32)))
            nbr_ref[:, k:k + 1] = jnp.clip(best_i, 0.0,
                                           float(_N - 1)).astype(jnp.int32)
            i_prev = best_i

    return _knn_body


def _knn(lo, hi, geom_rows, gt, brow, bcol, base, nrows):
    return pl.pallas_call(
        _make_knn_body(base),
        grid=(nrows // _RB,),
        in_specs=[
            pl.BlockSpec(memory_space=pltpu.SMEM),
            pl.BlockSpec(memory_space=pltpu.SMEM),
            pl.BlockSpec((_RB, 128), lambda i: (i, 0)),
            pl.BlockSpec((_NCHUNK, 128, _C), lambda i: (0, 0, 0)),
            pl.BlockSpec((_RB, 1), lambda i: (i, 0)),
            pl.BlockSpec((_NCHUNK, 1, _C), lambda i: (0, 0, 0)),
        ],
        out_specs=pl.BlockSpec((_RB, 128), lambda i: (i, 0)),
        out_shape=jax.ShapeDtypeStruct((nrows, 128), jnp.int32),
        scratch_shapes=[pltpu.VMEM((_NCHUNK, _RB, _C), jnp.float32)],
    )(lo, hi, geom_rows, gt, brow, bcol)


def _prep_body(g_ref, aux_ref, w1_ref, b1_ref, p_ref, tab_ref):
    wa = w1_ref[0:128, :]
    wb = w1_ref[128:256, :]
    g = g_ref[...]
    p_ref[...] = (jnp.dot(g, wa - wb, preferred_element_type=jnp.float32)
                  + b1_ref[...])
    q = jnp.dot(g, wb, preferred_element_type=jnp.float32)
    tab_ref[...] = jnp.concatenate(
        [q, aux_ref[...], jnp.zeros((_N, 112), jnp.float32)], axis=1)


def _prep(geom, aux, W1, b1r):
    return pl.pallas_call(
        _prep_body,
        out_shape=(jax.ShapeDtypeStruct((_N, 128), jnp.float32),
                   jax.ShapeDtypeStruct((_N, 256), jnp.float32)),
    )(geom, aux, W1, b1r)


def _sc_gather_call(tab, src, nch):
    mesh = plsc.VectorSubcoreMesh(core_axis_name="c", subcore_axis_name="s")

    @functools.partial(
        pl.kernel,
        mesh=mesh,
        out_type=jax.ShapeDtypeStruct((_NW, nch, _CB, 256), jnp.float32),
        scratch_types=[
            pltpu.VMEM((nch, _CB), jnp.int32),
            pltpu.VMEM((_CB, 256), jnp.float32),
            pltpu.VMEM((_CB, 256), jnp.float32),
            pltpu.SemaphoreType.DMA,
            pltpu.SemaphoreType.DMA,
        ],
    )
    def gather_kernel(tab_hbm, src_hbm, out_hbm, idx_v, buf0, buf1, sem0,
                      sem1):
        wid = lax.axis_index("s") * 2 + lax.axis_index("c")
        pltpu.sync_copy(src_hbm.at[wid], idx_v)
        bufs = (buf0, buf1)
        sems = (sem0, sem1)
        pltpu.async_copy(tab_hbm.at[idx_v.at[0]], buf0, sem0)

        @pl.loop(0, nch, step=2)
        def _(c):
            for b in range(2):
                ch = c + b
                nxt = ch + 1
                pltpu.make_async_copy(tab_hbm.at[idx_v.at[ch % nch]],
                                      bufs[b], sems[b]).wait()

                @pl.when(nxt < nch)
                def _():
                    pltpu.async_copy(tab_hbm.at[idx_v.at[nxt % nch]],
                                     bufs[1 - b], sems[1 - b])

                pltpu.sync_copy(bufs[b], out_hbm.at[wid, ch])

    return gather_kernel(tab, src)


def _stage1_body(qg_ref, p_ref, st_ref):
    i = pl.program_id(0)
    p = p_ref[...]
    s = jnp.zeros((1, 128), jnp.float32)
    ss = jnp.zeros((1, 128), jnp.float32)
    for k in range(_K):
        h = qg_ref[k] + p
        s = s + jnp.sum(h, axis=0, keepdims=True)
        ss = ss + jnp.sum(h * h, axis=0, keepdims=True)

    @pl.when(i == 0)
    def _():
        st_ref[...] = jnp.zeros_like(st_ref)

    st_ref[0:1, :] = st_ref[0:1, :] + s
    st_ref[1:2, :] = st_ref[1:2, :] + ss


def _stage1(eg, p, nblk):
    return pl.pallas_call(
        _stage1_body,
        grid=(nblk,),
        in_specs=[
            pl.BlockSpec((_K, _BN, 128), lambda i: (0, i, 0)),
            pl.BlockSpec((_BN, 128), lambda i: (i, 0)),
        ],
        out_specs=pl.BlockSpec((8, 128), lambda i: (0, 0)),
        out_shape=jax.ShapeDtypeStruct((8, 128), jnp.float32),
    )(eg, p)


def _stage2_body(qg_ref, p_ref, st1_ref, w2_ref, b2_ref, g1_ref, bb1_ref,
                 h2_ref, st2_ref):
    i = pl.program_id(0)
    inv_e = 1.0 / float(_E)
    m = st1_ref[0:1, :] * inv_e
    v = st1_ref[1:2, :] * inv_e - m * m
    sc = g1_ref[...] / jnp.sqrt(v + 1e-5)
    sh = bb1_ref[...] - m * sc
    p = p_ref[...]
    hcat = jnp.concatenate([qg_ref[k] + p for k in range(_K)], axis=0)
    h = jnp.maximum(hcat * sc + sh, 0.0)
    h2 = jnp.dot(h, w2_ref[...], preferred_element_type=jnp.float32) \
        + b2_ref[...]
    s = jnp.sum(h2, axis=0, keepdims=True)
    ss = jnp.sum(h2 * h2, axis=0, keepdims=True)
    for k in range(_K):
        h2_ref[k] = h2[k * _BN:(k + 1) * _BN, :]

    @pl.when(i == 0)
    def _():
        st2_ref[...] = jnp.zeros_like(st2_ref)

    st2_ref[0:1, :] = st2_ref[0:1, :] + s
    st2_ref[1:2, :] = st2_ref[1:2, :] + ss


def _stage2(eg, p, st1, W2, b2r, g1r, bb1r, nblk, npad):
    return pl.pallas_call(
        _stage2_body,
        grid=(nblk,),
        in_specs=[
            pl.BlockSpec((_K, _BN, 128), lambda i: (0, i, 0)),
            pl.BlockSpec((_BN, 128), lambda i: (i, 0)),
            pl.BlockSpec((8, 128), lambda i: (0, 0)),
            pl.BlockSpec((128, 128), lambda i: (0, 0)),
            pl.BlockSpec((1, 128), lambda i: (0, 0)),
            pl.BlockSpec((1, 128), lambda i: (0, 0)),
            pl.BlockSpec((1, 128), lambda i: (0, 0)),
        ],
        out_specs=(pl.BlockSpec((_K, _BN, 128), lambda i: (0, i, 0)),
                   pl.BlockSpec((8, 128), lambda i: (0, 0))),
        out_shape=(jax.ShapeDtypeStruct((_K, npad, 128), jnp.float32),
                   jax.ShapeDtypeStruct((8, 128), jnp.float32)),
    )(eg, p, st1, W2, b2r, g1r, bb1r)


def _stage3_body(h2_ref, ag_ref, aux_ref, st2_ref, wa1_ref, ba1_ref, wa2_ref,
                 ba2_ref, g2_ref, bb2_ref, lng_ref, lnb_ref, out_ref):
    inv_e = 1.0 / float(_E)
    m = st2_ref[0:1, :] * inv_e
    v = st2_ref[1:2, :] * inv_e - m * m
    sc2 = g2_ref[...] / jnp.sqrt(v + 1e-5)
    sh2 = bb2_ref[...] - m * sc2
    ai = aux_ref[...]
    ea = jnp.concatenate(
        [jnp.concatenate([ai, ag_ref[k][:, 0:16]], axis=1)
         for k in range(_K)], axis=0)
    a = jnp.maximum(
        jnp.dot(ea, wa1_ref[...], preferred_element_type=jnp.float32)
        + ba1_ref[...], 0.0)
    gb = jnp.dot(a, wa2_ref[...], preferred_element_type=jnp.float32) \
        + ba2_ref[...]
    o = jnp.full((_BN, 128), -jnp.inf, jnp.float32)
    for k in range(_K):
        ef = jnp.maximum(h2_ref[k] * sc2 + sh2, 0.0)
        gbk = gb[k * _BN:(k + 1) * _BN, :]
        gamma = 1.0 / (1.0 + jnp.exp(-(gbk[:, 0:128] + 1.0)))
        beta = gbk[:, 128:256]
        o = jnp.maximum(o, gamma * ef + beta)
    mu = jnp.mean(o, axis=1, keepdims=True)
    d = o - mu
    var = jnp.mean(d * d, axis=1, keepdims=True)
    out_ref[...] = jnp.maximum(
        d / jnp.sqrt(var + 1e-5) * lng_ref[...] + lnb_ref[...], 0.0)


def _stage3(h2, eg, aux_h, st2, Wa1, ba1r, Wa2, ba2r, g2r, bb2r, lngr, lnbr,
            nblk, nreal):
    return pl.pallas_call(
        _stage3_body,
        grid=(nblk,),
        in_specs=[
            pl.BlockSpec((_K, _BN, 128), lambda i: (0, i, 0)),
            pl.BlockSpec((_K, _BN, 128), lambda i: (0, i, 1)),
            pl.BlockSpec((_BN, 16), lambda i: (i, 0)),
            pl.BlockSpec((8, 128), lambda i: (0, 0)),
            pl.BlockSpec((32, 64), lambda i: (0, 0)),
            pl.BlockSpec((1, 64), lambda i: (0, 0)),
            pl.BlockSpec((64, 256), lambda i: (0, 0)),
            pl.BlockSpec((1, 256), lambda i: (0, 0)),
            pl.BlockSpec((1, 128), lambda i: (0, 0)),
            pl.BlockSpec((1, 128), lambda i: (0, 0)),
            pl.BlockSpec((1, 128), lambda i: (0, 0)),
            pl.BlockSpec((1, 128), lambda i: (0, 0)),
        ],
        out_specs=pl.BlockSpec((_BN, 128), lambda i: (i, 0)),
        out_shape=jax.ShapeDtypeStruct((nreal, 128), jnp.float32),
    )(h2, eg, aux_h, st2, Wa1, ba1r, Wa2, ba2r, g2r, bb2r, lngr, lnbr)


def _make_src(nbr, nreal, npad):
    pad_idx = jnp.broadcast_to(
        jnp.arange(npad - nreal, dtype=jnp.int32)[None, :],
        (_K, npad - nreal))
    src = jnp.concatenate([nbr[:, :_K].T, pad_idx], axis=1)
    return src.reshape(_NW, (npad * _K) // (_NW * _CB), _CB)


def kernel(geom, aux, batch, W1, b1, bn1_g, bn1_b, W2, b2, bn2_g, bn2_b,
           Wa1, ba1, Wa2, ba2, ln_g, ln_b):
    batch = batch.astype(jnp.int32)
    geom_pad = jnp.pad(geom, ((0, _NP - _N), (0, 0)))
    gt = geom_pad.T.reshape(128, _NCHUNK, _C).transpose(1, 0, 2)
    bcol = jnp.pad(batch, (0, _NP - _N),
                   constant_values=-2).reshape(_NCHUNK, 1, _C)
    brow_all = batch.reshape(_N, 1)

    def win(base, nrows):
        nblk = nrows // _RB
        first = batch[base + jnp.arange(nblk) * _RB]
        last = batch[base + (jnp.arange(nblk) + 1) * _RB - 1]
        lo = jnp.searchsorted(batch, first, side='left').astype(jnp.int32)
        hi = jnp.searchsorted(batch, last, side='right').astype(jnp.int32)
        return lo, hi

    lo1, hi1 = win(0, _N1)
    lo2, hi2 = win(_N1, _N2)

    nbr1 = _knn(lo1, hi1, geom[0:_N1], gt, brow_all[0:_N1], bcol, 0, _N1)
    nbr2 = _knn(lo2, hi2, geom[_N1:_N], gt, brow_all[_N1:_N], bcol, _N1, _N2)

    b1r = b1.reshape(1, 128)
    p, tab = _prep(geom, aux, W1, b1r)

    src1 = _make_src(nbr1, _N1, _NP1)
    src2 = _make_src(nbr2, _N2, _NP2)
    eg1 = _sc_gather_call(tab, src1, (_NP1 * _K) // (_NW * _CB))
    eg2 = _sc_gather_call(tab, src2, (_NP2 * _K) // (_NW * _CB))
    eg1 = eg1.reshape(_K, _NP1, 256)
    eg2 = eg2.reshape(_K, _NP2, 256)

    p1, p2 = p[0:_N1], p[_N1:_N]
    nblk1, nblk2 = _N1 // _BN, _N2 // _BN
    st1 = _stage1(eg1, p1, nblk1) + _stage1(eg2, p2, nblk2)
    b2r = b2.reshape(1, 128)
    g1r, bb1r = bn1_g.reshape(1, 128), bn1_b.reshape(1, 128)
    h2a, st2a = _stage2(eg1, p1, st1, W2, b2r, g1r, bb1r, nblk1, _NP1)
    h2b, st2b = _stage2(eg2, p2, st1, W2, b2r, g1r, bb1r, nblk2, _NP2)
    st2 = st2a + st2b

    s3 = (Wa1, ba1.reshape(1, 64), Wa2, ba2.reshape(1, 256),
          bn2_g.reshape(1, 128), bn2_b.reshape(1, 128),
          ln_g.reshape(1, 128), ln_b.reshape(1, 128))
    out1 = _stage3(h2a, eg1, aux[0:_N1], st2, *s3, nblk=nblk1, nreal=_N1)
    out2 = _stage3(h2b, eg2, aux[_N1:_N], st2, *s3, nblk=nblk2, nreal=_N2)
    return jnp.concatenate([out1, out2], axis=0)
